# Initial kernel scaffold; baseline (speedup 1.0000x reference)
import jax
import jax.numpy as jnp
from jax.experimental import pallas as pl


def kernel(src, table):
    raise NotImplementedError("write your pallas kernel here")


# speedup vs baseline: 1.6842x; 1.6842x over previous
import functools

import jax
import jax.numpy as jnp
from jax import lax
from jax.experimental import pallas as pl
from jax.experimental.pallas import tpu as pltpu
from jax.experimental.pallas import tpu_sc as plsc

EMB_DIM = 64
ROWS = 16384
COLS = 50

NC = 2
NS = 16
NW = NC * NS

CHUNK = 128
B_TOTAL = ROWS * COLS
B_PER_W = B_TOTAL // NW
N_CHUNKS = B_PER_W // CHUNK


def _emb_body(idx_hbm, table_hbm, out_hbm, idx_v, rows_v, sem):
    wid = lax.axis_index("s") * NC + lax.axis_index("c")
    base = wid * B_PER_W
    pltpu.sync_copy(idx_hbm.at[wid], idx_v)

    def chunk_body(j, carry):
        pltpu.async_copy(table_hbm.at[idx_v.at[j]], rows_v, sem).wait()
        pltpu.sync_copy(rows_v, out_hbm.at[pl.ds(base + j * CHUNK, CHUNK)])
        return carry

    lax.fori_loop(0, N_CHUNKS, chunk_body, 0)


def kernel(src, table):
    idx = src.astype(jnp.int32).reshape(NW, N_CHUNKS, CHUNK)
    mesh = plsc.VectorSubcoreMesh(core_axis_name="c", subcore_axis_name="s")
    emb = functools.partial(
        pl.kernel,
        mesh=mesh,
        out_type=jax.ShapeDtypeStruct((B_TOTAL, EMB_DIM), jnp.float32),
        scratch_types=[
            pltpu.VMEM((N_CHUNKS, CHUNK), jnp.int32),
            pltpu.VMEM((CHUNK, EMB_DIM), jnp.float32),
            pltpu.SemaphoreType.DMA,
        ],
        compiler_params=pltpu.CompilerParams(use_tc_tiling_on_sc=False),
    )(_emb_body)
    out = emb(idx, table)
    return out.reshape(ROWS, COLS, EMB_DIM)


# speedup vs baseline: 1.8719x; 1.1115x over previous
import functools

import jax
import jax.numpy as jnp
from jax import lax
from jax.experimental import pallas as pl
from jax.experimental.pallas import tpu as pltpu
from jax.experimental.pallas import tpu_sc as plsc

EMB_DIM = 64
ROWS = 16384
COLS = 50

NC = 2
NS = 16
NW = NC * NS

CHUNK = 128
B_TOTAL = ROWS * COLS
B_PER_W = B_TOTAL // NW
N_CHUNKS = B_PER_W // CHUNK

GPS = 4
S = GPS * CHUNK
NSUPER = B_PER_W // S
NBUF = 2


def _emb_body(idx_hbm, table_hbm, out_hbm,
              idx_v, buf0, buf1, gsem0, gsem1, ssem0, ssem1):
    bufs = (buf0, buf1)
    gsems = (gsem0, gsem1)
    ssems = (ssem0, ssem1)
    wid = lax.axis_index("s") * NC + lax.axis_index("c")
    base = wid * B_PER_W
    pltpu.sync_copy(idx_hbm.at[wid], idx_v)

    def fire_gathers(s, b):
        for q in range(GPS):
            pltpu.async_copy(
                table_hbm.at[idx_v.at[s * GPS + q]],
                bufs[b].at[pl.ds(q * CHUNK, CHUNK)],
                gsems[b])

    def drain_gathers(b):
        for q in range(GPS):
            pltpu.make_async_copy(
                table_hbm.at[idx_v.at[0]],
                bufs[b].at[pl.ds(q * CHUNK, CHUNK)],
                gsems[b]).wait()

    def fire_store(s, b):
        pltpu.async_copy(bufs[b], out_hbm.at[pl.ds(base + s * S, S)], ssems[b])

    def drain_store(b):
        pltpu.make_async_copy(bufs[b], out_hbm.at[pl.ds(base, S)], ssems[b]).wait()

    fire_gathers(0, 0)

    def outer(so, carry):
        for b in range(NBUF):
            s = so * NBUF + b
            drain_gathers(b)
            fire_store(s, b)

            @pl.when(s >= 1)
            def _():
                drain_store(1 - b)

            @pl.when(s + 1 < NSUPER)
            def _():
                fire_gathers(s + 1, 1 - b)

        return carry

    lax.fori_loop(0, NSUPER // NBUF, outer, 0)
    drain_store((NSUPER - 1) % NBUF)


def kernel(src, table):
    idx = src.astype(jnp.int32).reshape(NW, N_CHUNKS, CHUNK)
    mesh = plsc.VectorSubcoreMesh(core_axis_name="c", subcore_axis_name="s")
    emb = functools.partial(
        pl.kernel,
        mesh=mesh,
        out_type=jax.ShapeDtypeStruct((B_TOTAL, EMB_DIM), jnp.float32),
        scratch_types=[
            pltpu.VMEM((N_CHUNKS, CHUNK), jnp.int32),
            pltpu.VMEM((S, EMB_DIM), jnp.float32),
            pltpu.VMEM((S, EMB_DIM), jnp.float32),
            pltpu.SemaphoreType.DMA,
            pltpu.SemaphoreType.DMA,
            pltpu.SemaphoreType.DMA,
            pltpu.SemaphoreType.DMA,
        ],
        compiler_params=pltpu.CompilerParams(use_tc_tiling_on_sc=False),
    )(_emb_body)
    out = emb(idx, table)
    return out.reshape(ROWS, COLS, EMB_DIM)
